# baseline (speedup 1.0000x reference)
import functools

import jax
import jax.numpy as jnp
from jax import lax
from jax.experimental import pallas as pl
from jax.experimental.pallas import tpu as pltpu
from jax.experimental.pallas import tpu_sc as plsc

NN = 65536
EE = 1048576
CC = 64
GG = 2048
HALF = NN // 2
CH = 1024
NSUB = CH // 128
EW = EE // 16
NCHUNK = EW // CH
STRIPE = NN // 16


IDXB = 4096
NBLK = EW // IDXB
CPB = IDXB // CH
QF = 16


def _sc_segsum_body(m0, m1, m2, m3, src_h, dst_h, ea_h,
                    out0, out1, out2, out3,
                    src_a, dst_a, ea_a, src_b, dst_b, ea_b,
                    rows_a, rows_b,
                    accum, sem_a, sem_b, sem_sa, sem_sb, sem_i):
    cid = lax.axis_index("c")
    sid = lax.axis_index("s")
    zero16 = jnp.zeros((16,), jnp.float32)

    ibufs = ((src_a, dst_a, ea_a), (src_b, dst_b, ea_b))
    rbufs = ((rows_a, sem_a, sem_sa), (rows_b, sem_b, sem_sb))

    def fire_gather(mq0, mq1, src_v, off, rbuf):
        rows_v, sem = rbuf[0], rbuf[1]

        @pl.when(cid == 0)
        def _():
            for j in range(NSUB):
                pltpu.async_copy(mq0.at[src_v.at[pl.ds(off + j * 128, 128)]],
                                 rows_v.at[pl.ds(j * 128, 128)], sem)

        @pl.when(cid == 1)
        def _():
            for j in range(NSUB):
                pltpu.async_copy(mq1.at[src_v.at[pl.ds(off + j * 128, 128)]],
                                 rows_v.at[pl.ds(j * 128, 128)], sem)

    def drain_gather(rbuf):
        pltpu.make_async_copy(m0.at[pl.ds(0, CH)], rbuf[0], rbuf[1]).wait()

    def drain_scatter(rbuf):
        pltpu.make_async_copy(m0.at[pl.ds(0, CH)], rbuf[0], rbuf[2]).wait()

    def process(ibuf, c, rbuf):
        dst_v, ea_v = ibuf[1], ibuf[2]
        rows_v, _, sem_s = rbuf
        off = c * CH

        @plsc.parallel_loop(0, CH // 16, unroll=3)
        def _(g):
            ea16 = ea_v[pl.ds(off + g * 16, 16)]
            for l in range(16):
                sc = jnp.full((16,), ea16[l])
                r = g * 16 + l
                rows_v[r, 0:16] = rows_v[r, 0:16] * sc

        for j in range(NSUB):
            pltpu.async_copy(rows_v.at[pl.ds(j * 128, 128)],
                             accum.at[dst_v.at[c * NSUB + j]], sem_s, add=True)

    for p in range(2):
        mq0, mq1 = (m0, m1, m2, m3)[2 * p], (m0, m1, m2, m3)[2 * p + 1]
        oq0, oq1 = (out0, out1, out2, out3)[2 * p], (out0, out1, out2, out3)[2 * p + 1]

        @plsc.parallel_loop(0, CH, unroll=4)
        def _(r):
            rows_a[r, 0:16] = zero16

        for q in range(STRIPE // CH):
            pltpu.sync_copy(rows_a, accum.at[pl.ds(sid * STRIPE + q * CH, CH)])
        plsc.subcore_barrier()

        pltpu.sync_copy(src_h.at[pl.ds(sid * EW, IDXB)], ibufs[0][0])
        pltpu.sync_copy(dst_h.at[pl.ds(sid * (EW // 128), IDXB // 128)],
                        ibufs[0][1])
        pltpu.sync_copy(ea_h.at[pl.ds(sid * EW, IDXB)], ibufs[0][2])
        fire_gather(mq0, mq1, ibufs[0][0], 0, rbufs[0])

        def block(b, carry):
            nb = lax.rem(b + 1, NBLK)
            nbase = sid * EW + nb * IDXB

            def fire_iblock(ib):
                pltpu.async_copy(src_h.at[pl.ds(nbase, IDXB)], ib[0], sem_i)
                pltpu.async_copy(dst_h.at[pl.ds(nbase // 128, IDXB // 128)],
                                 ib[1], sem_i)
                pltpu.async_copy(ea_h.at[pl.ds(nbase, IDXB)], ib[2], sem_i)

            def run_block(ib_cur, ib_nxt):
                fire_iblock(ib_nxt)
                for c in range(CPB):
                    cur = rbufs[c % 2]
                    nxt = rbufs[(c + 1) % 2]
                    if c < CPB - 1:
                        if c == 0:
                            @pl.when(b > 0)
                            def _():
                                drain_scatter(nxt)
                        else:
                            drain_scatter(nxt)
                        fire_gather(mq0, mq1, ib_cur[0], (c + 1) * CH, nxt)
                    drain_gather(cur)
                    process(ib_cur, c, cur)
                pltpu.make_async_copy(src_h.at[pl.ds(0, IDXB)], ib_nxt[0],
                                      sem_i).wait()
                pltpu.make_async_copy(dst_h.at[pl.ds(0, IDXB // 128)],
                                      ib_nxt[1], sem_i).wait()
                pltpu.make_async_copy(ea_h.at[pl.ds(0, IDXB)], ib_nxt[2],
                                      sem_i).wait()
                drain_scatter(rbufs[(CPB - 2) % 2])
                fire_gather(mq0, mq1, ib_nxt[0], 0, rbufs[0])

            @pl.when(lax.rem(b, 2) == 0)
            def _():
                run_block(ibufs[0], ibufs[1])

            @pl.when(lax.rem(b, 2) == 1)
            def _():
                run_block(ibufs[1], ibufs[0])

            return carry

        lax.fori_loop(0, NBLK, block, 0)
        drain_gather(rbufs[0])
        drain_scatter(rbufs[(CPB - 1) % 2])
        plsc.subcore_barrier()

        @pl.when(cid == 0)
        def _():
            pltpu.sync_copy(accum.at[pl.ds(sid * STRIPE, STRIPE)],
                            oq0.at[pl.ds(sid * STRIPE, STRIPE)])

        @pl.when(cid == 1)
        def _():
            pltpu.sync_copy(accum.at[pl.ds(sid * STRIPE, STRIPE)],
                            oq1.at[pl.ds(sid * STRIPE, STRIPE)])

        plsc.subcore_barrier()


def _sc_segsum(m0, m1, m2, m3, src, dst2, ea):
    mesh = plsc.VectorSubcoreMesh(core_axis_name="c", subcore_axis_name="s")
    f = pl.kernel(
        _sc_segsum_body,
        out_type=tuple(jax.ShapeDtypeStruct((NN, QF), jnp.float32)
                       for _ in range(4)),
        mesh=mesh,
        scratch_types=(
            pltpu.VMEM((IDXB,), jnp.int32),
            pltpu.VMEM((IDXB // 128, 128), jnp.int32),
            pltpu.VMEM((IDXB,), jnp.float32),
            pltpu.VMEM((IDXB,), jnp.int32),
            pltpu.VMEM((IDXB // 128, 128), jnp.int32),
            pltpu.VMEM((IDXB,), jnp.float32),
            pltpu.VMEM((CH, QF), jnp.float32),
            pltpu.VMEM((CH, QF), jnp.float32),
            pltpu.VMEM_SHARED((NN, QF), jnp.float32),
            pltpu.SemaphoreType.DMA,
            pltpu.SemaphoreType.DMA,
            pltpu.SemaphoreType.DMA,
            pltpu.SemaphoreType.DMA,
            pltpu.SemaphoreType.DMA,
        ),
        compiler_params=pltpu.CompilerParams(use_tc_tiling_on_sc=False),
    )
    return f(m0, m1, m2, m3, src, dst2, ea)


BM = 2048


def _mm(a, b):
    return lax.dot_general(a, b, (((1,), (0,)), ((), ())),
                           preferred_element_type=jnp.float32)


def _gru_block(agg, h, wih, whh, bih, bhh):
    gi = _mm(agg, wih) + bih
    gh = _mm(h, whh) + bhh
    r = jax.nn.sigmoid(gi[:, 0:CC] + gh[:, 0:CC])
    z = jax.nn.sigmoid(gi[:, CC:2 * CC] + gh[:, CC:2 * CC])
    n = jnp.tanh(gi[:, 2 * CC:3 * CC] + r * gh[:, 2 * CC:3 * CC])
    return (1.0 - z) * n + z * h


def _k1_body(h_ref, w_ref, m0_ref, m1_ref, m2_ref, m3_ref):
    m = _mm(h_ref[...], w_ref[...])
    for q, ref in enumerate((m0_ref, m1_ref, m2_ref, m3_ref)):
        ref[...] = m[:, q * 16:(q + 1) * 16]


def _k1(h, w):
    return pl.pallas_call(
        _k1_body,
        grid=(NN // BM,),
        in_specs=[pl.BlockSpec((BM, CC), lambda i: (i, 0)),
                  pl.BlockSpec((CC, CC), lambda i: (0, 0))],
        out_specs=[pl.BlockSpec((BM, 16), lambda i: (i, 0))] * 4,
        out_shape=[jax.ShapeDtypeStruct((NN, 16), jnp.float32)] * 4,
    )(h, w)


def _k2_body(a0, a1, a2, a3, h, wih, whh, bih, bhh, w1,
             h1_ref, m0_ref, m1_ref, m2_ref, m3_ref):
    agg = jnp.concatenate([a0[...], a1[...], a2[...], a3[...]], axis=1)
    hn = _gru_block(agg, h[...], wih[...], whh[...], bih[...], bhh[...])
    h1_ref[...] = hn
    m = _mm(hn, w1[...])
    for q, ref in enumerate((m0_ref, m1_ref, m2_ref, m3_ref)):
        ref[...] = m[:, q * 16:(q + 1) * 16]


def _k2(a0, a1, a2, a3, h, wih, whh, bih, bhh, w1):
    return pl.pallas_call(
        _k2_body,
        grid=(NN // BM,),
        in_specs=[pl.BlockSpec((BM, 16), lambda i: (i, 0))] * 4 +
                 [pl.BlockSpec((BM, CC), lambda i: (i, 0)),
                  pl.BlockSpec((CC, 3 * CC), lambda i: (0, 0)),
                  pl.BlockSpec((CC, 3 * CC), lambda i: (0, 0)),
                  pl.BlockSpec((1, 3 * CC), lambda i: (0, 0)),
                  pl.BlockSpec((1, 3 * CC), lambda i: (0, 0)),
                  pl.BlockSpec((CC, CC), lambda i: (0, 0))],
        out_specs=[pl.BlockSpec((BM, CC), lambda i: (i, 0))] +
                  [pl.BlockSpec((BM, 16), lambda i: (i, 0))] * 4,
        out_shape=[jax.ShapeDtypeStruct((NN, CC), jnp.float32)] +
                  [jax.ShapeDtypeStruct((NN, 16), jnp.float32)] * 4,
    )(a0, a1, a2, a3, h, wih, whh, bih, bhh, w1)


def _k3_body(a0, a1, a2, a3, h, wih, whh, bih, bhh, cw, cb, c_ref):
    agg = jnp.concatenate([a0[...], a1[...], a2[...], a3[...]], axis=1)
    hn = _gru_block(agg, h[...], wih[...], whh[...], bih[...], bhh[...])
    g = jnp.maximum(hn, 0.0)
    c = jnp.sum(g * cw[...], axis=1) + cb[0, 0]
    c_ref[...] = jnp.maximum(c, 0.0)


def _k3(a0, a1, a2, a3, h, wih, whh, bih, bhh, cw, cb):
    return pl.pallas_call(
        _k3_body,
        grid=(NN // BM,),
        in_specs=[pl.BlockSpec((BM, 16), lambda i: (i, 0))] * 4 +
                 [pl.BlockSpec((BM, CC), lambda i: (i, 0)),
                  pl.BlockSpec((CC, 3 * CC), lambda i: (0, 0)),
                  pl.BlockSpec((CC, 3 * CC), lambda i: (0, 0)),
                  pl.BlockSpec((1, 3 * CC), lambda i: (0, 0)),
                  pl.BlockSpec((1, 3 * CC), lambda i: (0, 0)),
                  pl.BlockSpec((1, CC), lambda i: (0, 0)),
                  pl.BlockSpec((1, 1), lambda i: (0, 0))],
        out_specs=pl.BlockSpec((BM,), lambda i: (i,)),
        out_shape=jax.ShapeDtypeStruct((NN,), jnp.float32),
    )(a0, a1, a2, a3, h, wih, whh, bih, bhh, cw, cb)


def _k4_body(v, w1, b1, w2, b2, o_ref):
    t = jnp.maximum(_mm(v[...], w1[...]) + b1[...], 0.0)
    o = _mm(t, w2[...]) + b2[...]
    o_ref[...] = jax.nn.softmax(o, axis=-1)


def _k4(v, w1, b1, w2, b2):
    return pl.pallas_call(
        _k4_body,
        grid=(GG // 256,),
        in_specs=[pl.BlockSpec((256, 32), lambda i: (i, 0)),
                  pl.BlockSpec((32, CC), lambda i: (0, 0)),
                  pl.BlockSpec((1, CC), lambda i: (0, 0)),
                  pl.BlockSpec((CC, 10), lambda i: (0, 0)),
                  pl.BlockSpec((1, 10), lambda i: (0, 0))],
        out_specs=pl.BlockSpec((256, 10), lambda i: (i, 0)),
        out_shape=jax.ShapeDtypeStruct((GG, 10), jnp.float32),
    )(v, w1, b1, w2, b2)


def kernel(x, edge_index, edge_attr, batch, ggc_weight, gru_w_ih, gru_w_hh,
           gru_b_ih, gru_b_hh, cnn1_w, cnn1_b, lin1_w, lin1_b, lin2_w, lin2_b):
    src = edge_index[0]
    dst = edge_index[1]
    wih = gru_w_ih.T
    whh = gru_w_hh.T
    bih = gru_b_ih.reshape(1, 3 * CC)
    bhh = gru_b_hh.reshape(1, 3 * CC)
    cw = cnn1_w[:, :, 0]
    cb = cnn1_b.reshape(1, 1)
    w1t = lin1_w.T
    b1 = lin1_b.reshape(1, CC)
    w2t = lin2_w.T
    b2 = lin2_b.reshape(1, 10)

    dst2 = dst.reshape(EE // 128, 128)
    m0, m1, m2, m3 = _k1(x, ggc_weight[0])
    a0, a1, a2, a3 = _sc_segsum(m0, m1, m2, m3, src, dst2, edge_attr)
    h1, m0, m1, m2, m3 = _k2(a0, a1, a2, a3, x, wih, whh, bih, bhh,
                             ggc_weight[1])
    a0, a1, a2, a3 = _sc_segsum(m0, m1, m2, m3, src, dst2, edge_attr)
    c = _k3(a0, a1, a2, a3, h1, wih, whh, bih, bhh, cw, cb)
    v = c.reshape(GG, NN // GG)
    return _k4(v, w1t, b1, w2t, b2)


# speedup vs baseline: 1.0278x; 1.0278x over previous
import functools

import jax
import jax.numpy as jnp
from jax import lax
from jax.experimental import pallas as pl
from jax.experimental.pallas import tpu as pltpu
from jax.experimental.pallas import tpu_sc as plsc

NN = 65536
EE = 1048576
CC = 64
GG = 2048
HALF = NN // 2
CH = 1024
NSUB = CH // 128
EW = EE // 16
NCHUNK = EW // CH
STRIPE = NN // 16


IDXB = 4096
NBLK = EW // IDXB
CPB = IDXB // CH
QF = 16


def _sc_segsum_body(m0, m1, m2, m3, src_h, dst_h, ea_h,
                    out0, out1, out2, out3,
                    src_a, dst_a, ea_a, src_b, dst_b, ea_b,
                    rows_a, rows_b,
                    accum, sem_a, sem_b, sem_sa, sem_sb, sem_i):
    cid = lax.axis_index("c")
    sid = lax.axis_index("s")
    zero16 = jnp.zeros((16,), jnp.float32)

    ibufs = ((src_a, dst_a, ea_a), (src_b, dst_b, ea_b))
    rbufs = ((rows_a, sem_a, sem_sa), (rows_b, sem_b, sem_sb))

    def fire_gather(mq0, mq1, src_v, off, rbuf):
        rows_v, sem = rbuf[0], rbuf[1]

        @pl.when(cid == 0)
        def _():
            for j in range(NSUB):
                pltpu.async_copy(mq0.at[src_v.at[pl.ds(off + j * 128, 128)]],
                                 rows_v.at[pl.ds(j * 128, 128)], sem)

        @pl.when(cid == 1)
        def _():
            for j in range(NSUB):
                pltpu.async_copy(mq1.at[src_v.at[pl.ds(off + j * 128, 128)]],
                                 rows_v.at[pl.ds(j * 128, 128)], sem)

    def drain_gather(rbuf):
        pltpu.make_async_copy(m0.at[pl.ds(0, CH)], rbuf[0], rbuf[1]).wait()

    def drain_scatter(rbuf):
        pltpu.make_async_copy(m0.at[pl.ds(0, CH)], rbuf[0], rbuf[2]).wait()

    def process(ibuf, c, rbuf):
        dst_v, ea_v = ibuf[1], ibuf[2]
        rows_v, _, sem_s = rbuf
        off = c * CH

        @plsc.parallel_loop(0, CH // 16, unroll=2)
        def _(g):
            ea16 = ea_v[pl.ds(off + g * 16, 16)]
            for l in range(16):
                sc = jnp.full((16,), ea16[l])
                r = g * 16 + l
                rows_v[r, 0:16] = rows_v[r, 0:16] * sc

        for j in range(NSUB):
            pltpu.async_copy(rows_v.at[pl.ds(j * 128, 128)],
                             accum.at[dst_v.at[c * NSUB + j]], sem_s, add=True)

    for p in range(2):
        mq0, mq1 = (m0, m1, m2, m3)[2 * p], (m0, m1, m2, m3)[2 * p + 1]
        oq0, oq1 = (out0, out1, out2, out3)[2 * p], (out0, out1, out2, out3)[2 * p + 1]

        @plsc.parallel_loop(0, CH, unroll=4)
        def _(r):
            rows_a[r, 0:16] = zero16

        for q in range(STRIPE // CH):
            pltpu.sync_copy(rows_a, accum.at[pl.ds(sid * STRIPE + q * CH, CH)])
        plsc.subcore_barrier()

        pltpu.sync_copy(src_h.at[pl.ds(sid * EW, IDXB)], ibufs[0][0])
        pltpu.sync_copy(dst_h.at[pl.ds(sid * (EW // 128), IDXB // 128)],
                        ibufs[0][1])
        pltpu.sync_copy(ea_h.at[pl.ds(sid * EW, IDXB)], ibufs[0][2])
        fire_gather(mq0, mq1, ibufs[0][0], 0, rbufs[0])

        def block(b, carry):
            nb = lax.rem(b + 1, NBLK)
            nbase = sid * EW + nb * IDXB

            def fire_iblock(ib):
                pltpu.async_copy(src_h.at[pl.ds(nbase, IDXB)], ib[0], sem_i)
                pltpu.async_copy(dst_h.at[pl.ds(nbase // 128, IDXB // 128)],
                                 ib[1], sem_i)
                pltpu.async_copy(ea_h.at[pl.ds(nbase, IDXB)], ib[2], sem_i)

            def run_block(ib_cur, ib_nxt):
                fire_iblock(ib_nxt)
                for c in range(CPB):
                    cur = rbufs[c % 2]
                    nxt = rbufs[(c + 1) % 2]
                    if c < CPB - 1:
                        if c == 0:
                            @pl.when(b > 0)
                            def _():
                                drain_scatter(nxt)
                        else:
                            drain_scatter(nxt)
                        fire_gather(mq0, mq1, ib_cur[0], (c + 1) * CH, nxt)
                    drain_gather(cur)
                    process(ib_cur, c, cur)
                pltpu.make_async_copy(src_h.at[pl.ds(0, IDXB)], ib_nxt[0],
                                      sem_i).wait()
                pltpu.make_async_copy(dst_h.at[pl.ds(0, IDXB // 128)],
                                      ib_nxt[1], sem_i).wait()
                pltpu.make_async_copy(ea_h.at[pl.ds(0, IDXB)], ib_nxt[2],
                                      sem_i).wait()
                drain_scatter(rbufs[(CPB - 2) % 2])
                fire_gather(mq0, mq1, ib_nxt[0], 0, rbufs[0])

            @pl.when(lax.rem(b, 2) == 0)
            def _():
                run_block(ibufs[0], ibufs[1])

            @pl.when(lax.rem(b, 2) == 1)
            def _():
                run_block(ibufs[1], ibufs[0])

            return carry

        lax.fori_loop(0, NBLK, block, 0)
        drain_gather(rbufs[0])
        drain_scatter(rbufs[(CPB - 1) % 2])
        plsc.subcore_barrier()

        @pl.when(cid == 0)
        def _():
            pltpu.sync_copy(accum.at[pl.ds(sid * STRIPE, STRIPE)],
                            oq0.at[pl.ds(sid * STRIPE, STRIPE)])

        @pl.when(cid == 1)
        def _():
            pltpu.sync_copy(accum.at[pl.ds(sid * STRIPE, STRIPE)],
                            oq1.at[pl.ds(sid * STRIPE, STRIPE)])

        plsc.subcore_barrier()


def _sc_segsum(m0, m1, m2, m3, src, dst2, ea):
    mesh = plsc.VectorSubcoreMesh(core_axis_name="c", subcore_axis_name="s")
    f = pl.kernel(
        _sc_segsum_body,
        out_type=tuple(jax.ShapeDtypeStruct((NN, QF), jnp.float32)
                       for _ in range(4)),
        mesh=mesh,
        scratch_types=(
            pltpu.VMEM((IDXB,), jnp.int32),
            pltpu.VMEM((IDXB // 128, 128), jnp.int32),
            pltpu.VMEM((IDXB,), jnp.float32),
            pltpu.VMEM((IDXB,), jnp.int32),
            pltpu.VMEM((IDXB // 128, 128), jnp.int32),
            pltpu.VMEM((IDXB,), jnp.float32),
            pltpu.VMEM((CH, QF), jnp.float32),
            pltpu.VMEM((CH, QF), jnp.float32),
            pltpu.VMEM_SHARED((NN, QF), jnp.float32),
            pltpu.SemaphoreType.DMA,
            pltpu.SemaphoreType.DMA,
            pltpu.SemaphoreType.DMA,
            pltpu.SemaphoreType.DMA,
            pltpu.SemaphoreType.DMA,
        ),
        compiler_params=pltpu.CompilerParams(use_tc_tiling_on_sc=False),
    )
    return f(m0, m1, m2, m3, src, dst2, ea)


BM = 2048


def _mm(a, b):
    return lax.dot_general(a, b, (((1,), (0,)), ((), ())),
                           preferred_element_type=jnp.float32)


def _gru_block(agg, h, wih, whh, bih, bhh):
    gi = _mm(agg, wih) + bih
    gh = _mm(h, whh) + bhh
    r = jax.nn.sigmoid(gi[:, 0:CC] + gh[:, 0:CC])
    z = jax.nn.sigmoid(gi[:, CC:2 * CC] + gh[:, CC:2 * CC])
    n = jnp.tanh(gi[:, 2 * CC:3 * CC] + r * gh[:, 2 * CC:3 * CC])
    return (1.0 - z) * n + z * h


def _k1_body(h_ref, w_ref, m0_ref, m1_ref, m2_ref, m3_ref):
    m = _mm(h_ref[...], w_ref[...])
    for q, ref in enumerate((m0_ref, m1_ref, m2_ref, m3_ref)):
        ref[...] = m[:, q * 16:(q + 1) * 16]


def _k1(h, w):
    return pl.pallas_call(
        _k1_body,
        grid=(NN // BM,),
        in_specs=[pl.BlockSpec((BM, CC), lambda i: (i, 0)),
                  pl.BlockSpec((CC, CC), lambda i: (0, 0))],
        out_specs=[pl.BlockSpec((BM, 16), lambda i: (i, 0))] * 4,
        out_shape=[jax.ShapeDtypeStruct((NN, 16), jnp.float32)] * 4,
    )(h, w)


def _k2_body(a0, a1, a2, a3, h, wih, whh, bih, bhh, w1,
             h1_ref, m0_ref, m1_ref, m2_ref, m3_ref):
    agg = jnp.concatenate([a0[...], a1[...], a2[...], a3[...]], axis=1)
    hn = _gru_block(agg, h[...], wih[...], whh[...], bih[...], bhh[...])
    h1_ref[...] = hn
    m = _mm(hn, w1[...])
    for q, ref in enumerate((m0_ref, m1_ref, m2_ref, m3_ref)):
        ref[...] = m[:, q * 16:(q + 1) * 16]


def _k2(a0, a1, a2, a3, h, wih, whh, bih, bhh, w1):
    return pl.pallas_call(
        _k2_body,
        grid=(NN // BM,),
        in_specs=[pl.BlockSpec((BM, 16), lambda i: (i, 0))] * 4 +
                 [pl.BlockSpec((BM, CC), lambda i: (i, 0)),
                  pl.BlockSpec((CC, 3 * CC), lambda i: (0, 0)),
                  pl.BlockSpec((CC, 3 * CC), lambda i: (0, 0)),
                  pl.BlockSpec((1, 3 * CC), lambda i: (0, 0)),
                  pl.BlockSpec((1, 3 * CC), lambda i: (0, 0)),
                  pl.BlockSpec((CC, CC), lambda i: (0, 0))],
        out_specs=[pl.BlockSpec((BM, CC), lambda i: (i, 0))] +
                  [pl.BlockSpec((BM, 16), lambda i: (i, 0))] * 4,
        out_shape=[jax.ShapeDtypeStruct((NN, CC), jnp.float32)] +
                  [jax.ShapeDtypeStruct((NN, 16), jnp.float32)] * 4,
    )(a0, a1, a2, a3, h, wih, whh, bih, bhh, w1)


def _k3_body(a0, a1, a2, a3, h, wih, whh, bih, bhh, cw, cb, c_ref):
    agg = jnp.concatenate([a0[...], a1[...], a2[...], a3[...]], axis=1)
    hn = _gru_block(agg, h[...], wih[...], whh[...], bih[...], bhh[...])
    g = jnp.maximum(hn, 0.0)
    c = jnp.sum(g * cw[...], axis=1) + cb[0, 0]
    c_ref[...] = jnp.maximum(c, 0.0)


def _k3(a0, a1, a2, a3, h, wih, whh, bih, bhh, cw, cb):
    return pl.pallas_call(
        _k3_body,
        grid=(NN // BM,),
        in_specs=[pl.BlockSpec((BM, 16), lambda i: (i, 0))] * 4 +
                 [pl.BlockSpec((BM, CC), lambda i: (i, 0)),
                  pl.BlockSpec((CC, 3 * CC), lambda i: (0, 0)),
                  pl.BlockSpec((CC, 3 * CC), lambda i: (0, 0)),
                  pl.BlockSpec((1, 3 * CC), lambda i: (0, 0)),
                  pl.BlockSpec((1, 3 * CC), lambda i: (0, 0)),
                  pl.BlockSpec((1, CC), lambda i: (0, 0)),
                  pl.BlockSpec((1, 1), lambda i: (0, 0))],
        out_specs=pl.BlockSpec((BM,), lambda i: (i,)),
        out_shape=jax.ShapeDtypeStruct((NN,), jnp.float32),
    )(a0, a1, a2, a3, h, wih, whh, bih, bhh, cw, cb)


def _k4_body(v, w1, b1, w2, b2, o_ref):
    t = jnp.maximum(_mm(v[...], w1[...]) + b1[...], 0.0)
    o = _mm(t, w2[...]) + b2[...]
    o_ref[...] = jax.nn.softmax(o, axis=-1)


def _k4(v, w1, b1, w2, b2):
    return pl.pallas_call(
        _k4_body,
        grid=(GG // 256,),
        in_specs=[pl.BlockSpec((256, 32), lambda i: (i, 0)),
                  pl.BlockSpec((32, CC), lambda i: (0, 0)),
                  pl.BlockSpec((1, CC), lambda i: (0, 0)),
                  pl.BlockSpec((CC, 10), lambda i: (0, 0)),
                  pl.BlockSpec((1, 10), lambda i: (0, 0))],
        out_specs=pl.BlockSpec((256, 10), lambda i: (i, 0)),
        out_shape=jax.ShapeDtypeStruct((GG, 10), jnp.float32),
    )(v, w1, b1, w2, b2)


def kernel(x, edge_index, edge_attr, batch, ggc_weight, gru_w_ih, gru_w_hh,
           gru_b_ih, gru_b_hh, cnn1_w, cnn1_b, lin1_w, lin1_b, lin2_w, lin2_b):
    src = edge_index[0]
    dst = edge_index[1]
    wih = gru_w_ih.T
    whh = gru_w_hh.T
    bih = gru_b_ih.reshape(1, 3 * CC)
    bhh = gru_b_hh.reshape(1, 3 * CC)
    cw = cnn1_w[:, :, 0]
    cb = cnn1_b.reshape(1, 1)
    w1t = lin1_w.T
    b1 = lin1_b.reshape(1, CC)
    w2t = lin2_w.T
    b2 = lin2_b.reshape(1, 10)

    dst2 = dst.reshape(EE // 128, 128)
    m0, m1, m2, m3 = _k1(x, ggc_weight[0])
    a0, a1, a2, a3 = _sc_segsum(m0, m1, m2, m3, src, dst2, edge_attr)
    h1, m0, m1, m2, m3 = _k2(a0, a1, a2, a3, x, wih, whh, bih, bhh,
                             ggc_weight[1])
    a0, a1, a2, a3 = _sc_segsum(m0, m1, m2, m3, src, dst2, edge_attr)
    c = _k3(a0, a1, a2, a3, h1, wih, whh, bih, bhh, cw, cb)
    v = c.reshape(GG, NN // GG)
    return _k4(v, w1t, b1, w2t, b2)


# speedup vs baseline: 1.0440x; 1.0158x over previous
import functools

import jax
import jax.numpy as jnp
from jax import lax
from jax.experimental import pallas as pl
from jax.experimental.pallas import tpu as pltpu
from jax.experimental.pallas import tpu_sc as plsc

NN = 65536
EE = 1048576
CC = 64
GG = 2048
HALF = NN // 2
CH = 1024
NSUB = CH // 128
EW = EE // 16
NCHUNK = EW // CH
STRIPE = NN // 16


IDXB = 4096
NBLK = EW // IDXB
CPB = IDXB // CH
QF = 16


def _sc_segsum_body(m0, m1, m2, m3, src_h, dst_h, ea_h,
                    out0, out1, out2, out3,
                    src_a, dst_a, ea_a, src_b, dst_b, ea_b,
                    rows_a, rows_b,
                    accum, sem_a, sem_b, sem_sa, sem_sb, sem_i):
    cid = lax.axis_index("c")
    sid = lax.axis_index("s")
    zero16 = jnp.zeros((16,), jnp.float32)

    ibufs = ((src_a, dst_a, ea_a), (src_b, dst_b, ea_b))
    rbufs = ((rows_a, sem_a, sem_sa), (rows_b, sem_b, sem_sb))

    def fire_gather(mq0, mq1, src_v, off, rbuf):
        rows_v, sem = rbuf[0], rbuf[1]

        @pl.when(cid == 0)
        def _():
            for j in range(NSUB):
                pltpu.async_copy(mq0.at[src_v.at[pl.ds(off + j * 128, 128)]],
                                 rows_v.at[pl.ds(j * 128, 128)], sem)

        @pl.when(cid == 1)
        def _():
            for j in range(NSUB):
                pltpu.async_copy(mq1.at[src_v.at[pl.ds(off + j * 128, 128)]],
                                 rows_v.at[pl.ds(j * 128, 128)], sem)

    def drain_gather(rbuf):
        pltpu.make_async_copy(m0.at[pl.ds(0, CH)], rbuf[0], rbuf[1]).wait()

    def drain_scatter(rbuf):
        pltpu.make_async_copy(m0.at[pl.ds(0, CH)], rbuf[0], rbuf[2]).wait()

    def process(ibuf, c, rbuf):
        dst_v, ea_v = ibuf[1], ibuf[2]
        rows_v, _, sem_s = rbuf
        off = c * CH

        @plsc.parallel_loop(0, CH // 16, unroll=2)
        def _(g):
            ea16 = ea_v[pl.ds(off + g * 16, 16)]
            for l in range(16):
                sc = jnp.full((16,), ea16[l])
                r = g * 16 + l
                rows_v[r, 0:16] = rows_v[r, 0:16] * sc

        for j in range(NSUB):
            pltpu.async_copy(rows_v.at[pl.ds(j * 128, 128)],
                             accum.at[dst_v.at[c * NSUB + j]], sem_s, add=True)

    for p in range(2):
        mq0, mq1 = (m0, m1, m2, m3)[2 * p], (m0, m1, m2, m3)[2 * p + 1]
        oq0, oq1 = (out0, out1, out2, out3)[2 * p], (out0, out1, out2, out3)[2 * p + 1]

        @plsc.parallel_loop(0, CH, unroll=4)
        def _(r):
            rows_a[r, 0:16] = zero16

        for q in range(STRIPE // CH):
            pltpu.sync_copy(rows_a, accum.at[pl.ds(sid * STRIPE + q * CH, CH)])
        plsc.subcore_barrier()

        pltpu.sync_copy(src_h.at[pl.ds(sid * EW, IDXB)], ibufs[0][0])
        pltpu.sync_copy(dst_h.at[pl.ds(sid * (EW // 128), IDXB // 128)],
                        ibufs[0][1])
        pltpu.sync_copy(ea_h.at[pl.ds(sid * EW, IDXB)], ibufs[0][2])
        fire_gather(mq0, mq1, ibufs[0][0], 0, rbufs[0])

        def block(b, carry):
            nb = lax.rem(b + 1, NBLK)
            nbase = sid * EW + nb * IDXB

            def fire_iblock(ib):
                pltpu.async_copy(src_h.at[pl.ds(nbase, IDXB)], ib[0], sem_i)
                pltpu.async_copy(dst_h.at[pl.ds(nbase // 128, IDXB // 128)],
                                 ib[1], sem_i)
                pltpu.async_copy(ea_h.at[pl.ds(nbase, IDXB)], ib[2], sem_i)

            def run_block(ib_cur, ib_nxt):
                fire_iblock(ib_nxt)
                for c in range(CPB):
                    cur = rbufs[c % 2]
                    nxt = rbufs[(c + 1) % 2]
                    if c < CPB - 1:
                        if c == 0:
                            @pl.when(b > 0)
                            def _():
                                drain_scatter(nxt)
                        else:
                            drain_scatter(nxt)
                        fire_gather(mq0, mq1, ib_cur[0], (c + 1) * CH, nxt)
                    drain_gather(cur)
                    process(ib_cur, c, cur)
                pltpu.make_async_copy(src_h.at[pl.ds(0, IDXB)], ib_nxt[0],
                                      sem_i).wait()
                pltpu.make_async_copy(dst_h.at[pl.ds(0, IDXB // 128)],
                                      ib_nxt[1], sem_i).wait()
                pltpu.make_async_copy(ea_h.at[pl.ds(0, IDXB)], ib_nxt[2],
                                      sem_i).wait()
                drain_scatter(rbufs[(CPB - 2) % 2])
                fire_gather(mq0, mq1, ib_nxt[0], 0, rbufs[0])

            @pl.when(lax.rem(b, 2) == 0)
            def _():
                run_block(ibufs[0], ibufs[1])

            @pl.when(lax.rem(b, 2) == 1)
            def _():
                run_block(ibufs[1], ibufs[0])

            return carry

        lax.fori_loop(0, NBLK, block, 0)
        drain_gather(rbufs[0])
        drain_scatter(rbufs[(CPB - 1) % 2])
        plsc.subcore_barrier()

        @pl.when(cid == 0)
        def _():
            pltpu.sync_copy(accum.at[pl.ds(sid * STRIPE, STRIPE)],
                            oq0.at[pl.ds(sid * STRIPE, STRIPE)])

        @pl.when(cid == 1)
        def _():
            pltpu.sync_copy(accum.at[pl.ds(sid * STRIPE, STRIPE)],
                            oq1.at[pl.ds(sid * STRIPE, STRIPE)])

        plsc.subcore_barrier()


def _sc_segsum(m0, m1, m2, m3, src, dst2, ea):
    mesh = plsc.VectorSubcoreMesh(core_axis_name="c", subcore_axis_name="s")
    f = pl.kernel(
        _sc_segsum_body,
        out_type=tuple(jax.ShapeDtypeStruct((NN, QF), jnp.float32)
                       for _ in range(4)),
        mesh=mesh,
        scratch_types=(
            pltpu.VMEM((IDXB,), jnp.int32),
            pltpu.VMEM((IDXB // 128, 128), jnp.int32),
            pltpu.VMEM((IDXB,), jnp.float32),
            pltpu.VMEM((IDXB,), jnp.int32),
            pltpu.VMEM((IDXB // 128, 128), jnp.int32),
            pltpu.VMEM((IDXB,), jnp.float32),
            pltpu.VMEM((CH, QF), jnp.float32),
            pltpu.VMEM((CH, QF), jnp.float32),
            pltpu.VMEM_SHARED((NN, QF), jnp.float32),
            pltpu.SemaphoreType.DMA,
            pltpu.SemaphoreType.DMA,
            pltpu.SemaphoreType.DMA,
            pltpu.SemaphoreType.DMA,
            pltpu.SemaphoreType.DMA,
        ),
        compiler_params=pltpu.CompilerParams(use_tc_tiling_on_sc=False),
    )
    return f(m0, m1, m2, m3, src, dst2, ea)


BM = 4096


def _mm(a, b):
    return lax.dot_general(a, b, (((1,), (0,)), ((), ())),
                           preferred_element_type=jnp.float32)


def _gru_block(agg, h, wih, whh, bih, bhh):
    gi = _mm(agg, wih) + bih
    gh = _mm(h, whh) + bhh
    r = jax.nn.sigmoid(gi[:, 0:CC] + gh[:, 0:CC])
    z = jax.nn.sigmoid(gi[:, CC:2 * CC] + gh[:, CC:2 * CC])
    n = jnp.tanh(gi[:, 2 * CC:3 * CC] + r * gh[:, 2 * CC:3 * CC])
    return (1.0 - z) * n + z * h


def _k1_body(h_ref, w_ref, m0_ref, m1_ref, m2_ref, m3_ref):
    m = _mm(h_ref[...], w_ref[...])
    for q, ref in enumerate((m0_ref, m1_ref, m2_ref, m3_ref)):
        ref[...] = m[:, q * 16:(q + 1) * 16]


def _k1(h, w):
    return pl.pallas_call(
        _k1_body,
        grid=(NN // BM,),
        in_specs=[pl.BlockSpec((BM, CC), lambda i: (i, 0)),
                  pl.BlockSpec((CC, CC), lambda i: (0, 0))],
        out_specs=[pl.BlockSpec((BM, 16), lambda i: (i, 0))] * 4,
        out_shape=[jax.ShapeDtypeStruct((NN, 16), jnp.float32)] * 4,
    )(h, w)


def _k2_body(a0, a1, a2, a3, h, wih, whh, bih, bhh, w1,
             h1_ref, m0_ref, m1_ref, m2_ref, m3_ref):
    agg = jnp.concatenate([a0[...], a1[...], a2[...], a3[...]], axis=1)
    hn = _gru_block(agg, h[...], wih[...], whh[...], bih[...], bhh[...])
    h1_ref[...] = hn
    m = _mm(hn, w1[...])
    for q, ref in enumerate((m0_ref, m1_ref, m2_ref, m3_ref)):
        ref[...] = m[:, q * 16:(q + 1) * 16]


def _k2(a0, a1, a2, a3, h, wih, whh, bih, bhh, w1):
    return pl.pallas_call(
        _k2_body,
        grid=(NN // BM,),
        in_specs=[pl.BlockSpec((BM, 16), lambda i: (i, 0))] * 4 +
                 [pl.BlockSpec((BM, CC), lambda i: (i, 0)),
                  pl.BlockSpec((CC, 3 * CC), lambda i: (0, 0)),
                  pl.BlockSpec((CC, 3 * CC), lambda i: (0, 0)),
                  pl.BlockSpec((1, 3 * CC), lambda i: (0, 0)),
                  pl.BlockSpec((1, 3 * CC), lambda i: (0, 0)),
                  pl.BlockSpec((CC, CC), lambda i: (0, 0))],
        out_specs=[pl.BlockSpec((BM, CC), lambda i: (i, 0))] +
                  [pl.BlockSpec((BM, 16), lambda i: (i, 0))] * 4,
        out_shape=[jax.ShapeDtypeStruct((NN, CC), jnp.float32)] +
                  [jax.ShapeDtypeStruct((NN, 16), jnp.float32)] * 4,
    )(a0, a1, a2, a3, h, wih, whh, bih, bhh, w1)


def _k3_body(a0, a1, a2, a3, h, wih, whh, bih, bhh, cw, cb, c_ref):
    agg = jnp.concatenate([a0[...], a1[...], a2[...], a3[...]], axis=1)
    hn = _gru_block(agg, h[...], wih[...], whh[...], bih[...], bhh[...])
    g = jnp.maximum(hn, 0.0)
    c = jnp.sum(g * cw[...], axis=1) + cb[0, 0]
    c_ref[...] = jnp.maximum(c, 0.0)


def _k3(a0, a1, a2, a3, h, wih, whh, bih, bhh, cw, cb):
    return pl.pallas_call(
        _k3_body,
        grid=(NN // BM,),
        in_specs=[pl.BlockSpec((BM, 16), lambda i: (i, 0))] * 4 +
                 [pl.BlockSpec((BM, CC), lambda i: (i, 0)),
                  pl.BlockSpec((CC, 3 * CC), lambda i: (0, 0)),
                  pl.BlockSpec((CC, 3 * CC), lambda i: (0, 0)),
                  pl.BlockSpec((1, 3 * CC), lambda i: (0, 0)),
                  pl.BlockSpec((1, 3 * CC), lambda i: (0, 0)),
                  pl.BlockSpec((1, CC), lambda i: (0, 0)),
                  pl.BlockSpec((1, 1), lambda i: (0, 0))],
        out_specs=pl.BlockSpec((BM,), lambda i: (i,)),
        out_shape=jax.ShapeDtypeStruct((NN,), jnp.float32),
    )(a0, a1, a2, a3, h, wih, whh, bih, bhh, cw, cb)


def _k4_body(v, w1, b1, w2, b2, o_ref):
    t = jnp.maximum(_mm(v[...], w1[...]) + b1[...], 0.0)
    o = _mm(t, w2[...]) + b2[...]
    o_ref[...] = jax.nn.softmax(o, axis=-1)


def _k4(v, w1, b1, w2, b2):
    return pl.pallas_call(
        _k4_body,
        grid=(GG // 256,),
        in_specs=[pl.BlockSpec((256, 32), lambda i: (i, 0)),
                  pl.BlockSpec((32, CC), lambda i: (0, 0)),
                  pl.BlockSpec((1, CC), lambda i: (0, 0)),
                  pl.BlockSpec((CC, 10), lambda i: (0, 0)),
                  pl.BlockSpec((1, 10), lambda i: (0, 0))],
        out_specs=pl.BlockSpec((256, 10), lambda i: (i, 0)),
        out_shape=jax.ShapeDtypeStruct((GG, 10), jnp.float32),
    )(v, w1, b1, w2, b2)


def kernel(x, edge_index, edge_attr, batch, ggc_weight, gru_w_ih, gru_w_hh,
           gru_b_ih, gru_b_hh, cnn1_w, cnn1_b, lin1_w, lin1_b, lin2_w, lin2_b):
    src = edge_index[0]
    dst = edge_index[1]
    wih = gru_w_ih.T
    whh = gru_w_hh.T
    bih = gru_b_ih.reshape(1, 3 * CC)
    bhh = gru_b_hh.reshape(1, 3 * CC)
    cw = cnn1_w[:, :, 0]
    cb = cnn1_b.reshape(1, 1)
    w1t = lin1_w.T
    b1 = lin1_b.reshape(1, CC)
    w2t = lin2_w.T
    b2 = lin2_b.reshape(1, 10)

    dst2 = dst.reshape(EE // 128, 128)
    m0, m1, m2, m3 = _k1(x, ggc_weight[0])
    a0, a1, a2, a3 = _sc_segsum(m0, m1, m2, m3, src, dst2, edge_attr)
    h1, m0, m1, m2, m3 = _k2(a0, a1, a2, a3, x, wih, whh, bih, bhh,
                             ggc_weight[1])
    a0, a1, a2, a3 = _sc_segsum(m0, m1, m2, m3, src, dst2, edge_attr)
    c = _k3(a0, a1, a2, a3, h1, wih, whh, bih, bhh, cw, cb)
    v = c.reshape(GG, NN // GG)
    return _k4(v, w1t, b1, w2t, b2)
